# baseline (speedup 1.0000x reference)
import functools

import jax
import jax.numpy as jnp
from jax import lax
from jax.experimental import pallas as pl
from jax.experimental.pallas import tpu as pltpu
from jax.experimental.pallas import tpu_sc as plsc

_NC, _NS, _L = 2, 16, 16


def _make_bias_body(max_c, max_n, emb):
    nvec = emb // _L

    def _body(ts_ref, ch_ref, pw_hbm, cw_hbm, o_hbm, tsv, chv, pwv, cwv,
              ov):
        wid = lax.axis_index("s") * _NC + lax.axis_index("c")

        @pl.when(wid < max_c)
        def _():
            pltpu.sync_copy(ts_ref, tsv)
            pltpu.sync_copy(ch_ref, chv)
            pltpu.sync_copy(pw_hbm, pwv)
            pltpu.sync_copy(cw_hbm, cwv)
            tsvec = tsv[...]
            chvec0 = chv[pl.ds(0, _L)]
            chvec1 = chv[pl.ds(_L, _L)]
            n_tok = sum(tsvec[i] for i in range(_L))
            n_ch = sum(chvec0[i] + chvec1[i] for i in range(_L))
            cidx = jnp.where(wid < n_ch, wid, max_c - 1)
            for j in range(nvec):
                sl = pl.ds(j * _L, _L)
                cvec = cwv[cidx, sl]
                for n in range(max_n):
                    pidx = jnp.where(n < n_tok, n, max_n - 1)
                    ov[n, sl] = cvec + pwv[pidx, sl]
            pltpu.sync_copy(ov, o_hbm.at[wid])

    return _body


def _make_stream_body(bs, bb, ring):
    nb = bs // bb

    def _body(b_ref, x_hbm, o_hbm, xbuf, obuf, in_sems, out_sems):
        def in_copy(i, k):
            return pltpu.make_async_copy(
                x_hbm.at[pl.ds(i * bb, bb)], xbuf.at[k], in_sems.at[k])

        def out_copy(i, k):
            return pltpu.make_async_copy(
                obuf.at[k], o_hbm.at[pl.ds(i * bb, bb)], out_sems.at[k])

        bias = b_ref[...][None]
        for i in range(min(ring, nb)):
            in_copy(i, i % ring).start()
        for i in range(nb):
            k = i % ring
            in_copy(i, k).wait()
            if i >= ring:
                out_copy(i - ring, k).wait()
            obuf[k] = xbuf[k] + bias
            out_copy(i, k).start()
            if i + ring < nb:
                in_copy(i + ring, k).start()
        for i in range(max(nb - ring, 0), nb):
            out_copy(i, i % ring).wait()

    return _body


@functools.partial(jax.jit, static_argnames=("bb", "ring"))
def _run(x, ts_i, ch_i, patch_pos_w, ch_pos_w, bb=8, ring=16):
    bs, max_c, max_n, emb = x.shape
    bias_fn = pl.kernel(
        _make_bias_body(max_c, max_n, emb),
        out_type=jax.ShapeDtypeStruct((max_c, max_n, emb), x.dtype),
        mesh=plsc.VectorSubcoreMesh(core_axis_name="c", subcore_axis_name="s"),
        scratch_types=[
            pltpu.VMEM((_L,), jnp.int32),
            pltpu.VMEM((2 * _L,), jnp.int32),
            pltpu.VMEM((max_n, emb), x.dtype),
            pltpu.VMEM((max_c, emb), x.dtype),
            pltpu.VMEM((max_n, emb), x.dtype),
        ],
    )
    bias = bias_fn(ts_i, ch_i, patch_pos_w, ch_pos_w)
    out = pl.pallas_call(
        _make_stream_body(bs, bb, ring),
        in_specs=[
            pl.BlockSpec(memory_space=pltpu.VMEM),
            pl.BlockSpec(memory_space=pl.ANY),
        ],
        out_specs=pl.BlockSpec(memory_space=pl.ANY),
        out_shape=jax.ShapeDtypeStruct((bs, max_c, max_n, emb), x.dtype),
        scratch_shapes=[
            pltpu.VMEM((ring, bb, max_c, max_n, emb), x.dtype),
            pltpu.VMEM((ring, bb, max_c, max_n, emb), x.dtype),
            pltpu.SemaphoreType.DMA((ring,)),
            pltpu.SemaphoreType.DMA((ring,)),
        ],
    )(bias, x)
    return out


def kernel(x, ts_token_mask, ch_mask, patch_pos_w, ch_pos_w):
    ts_i = jnp.pad(ts_token_mask.astype(jnp.int32),
                   ((0, 0), (0, _L - ts_token_mask.shape[1]))).reshape(_L)
    ch_i = jnp.pad(ch_mask.astype(jnp.int32),
                   ((0, 0), (0, 2 * _L - ch_mask.shape[1]))).reshape(2 * _L)
    return _run(x, ts_i, ch_i, patch_pos_w, ch_pos_w)


# speedup vs baseline: 1.0046x; 1.0046x over previous
import functools

import jax
import jax.numpy as jnp
from jax import lax
from jax.experimental import pallas as pl
from jax.experimental.pallas import tpu as pltpu
from jax.experimental.pallas import tpu_sc as plsc

_NC, _NS, _L = 2, 16, 16


def _make_bias_body(max_c, max_n, emb):
    nvec = emb // _L

    def _body(ts_ref, ch_ref, pw_hbm, cw_hbm, o_hbm, tsv, chv, pwv, cwv,
              ov):
        wid = lax.axis_index("s") * _NC + lax.axis_index("c")

        @pl.when(wid < max_c)
        def _():
            pltpu.sync_copy(ts_ref, tsv)
            pltpu.sync_copy(ch_ref, chv)
            pltpu.sync_copy(pw_hbm, pwv)
            pltpu.sync_copy(cw_hbm, cwv)
            tsvec = tsv[...]
            chvec0 = chv[pl.ds(0, _L)]
            chvec1 = chv[pl.ds(_L, _L)]
            n_tok = sum(tsvec[i] for i in range(_L))
            n_ch = sum(chvec0[i] + chvec1[i] for i in range(_L))
            cidx = jnp.where(wid < n_ch, wid, max_c - 1)
            for j in range(nvec):
                sl = pl.ds(j * _L, _L)
                cvec = cwv[cidx, sl]
                for n in range(max_n):
                    pidx = jnp.where(n < n_tok, n, max_n - 1)
                    ov[n, sl] = cvec + pwv[pidx, sl]
            pltpu.sync_copy(ov, o_hbm.at[wid])

    return _body


def _make_stream_body(bs, bb, ring):
    nb = bs // bb

    def _body(b_ref, x_hbm, o_hbm, xbuf, obuf, in_sems, out_sems):
        def in_copy(i, k):
            return pltpu.make_async_copy(
                x_hbm.at[pl.ds(i * bb, bb)], xbuf.at[k], in_sems.at[k])

        def out_copy(i, k):
            return pltpu.make_async_copy(
                obuf.at[k], o_hbm.at[pl.ds(i * bb, bb)], out_sems.at[k])

        bias = b_ref[...][None]
        for i in range(min(ring, nb)):
            in_copy(i, i % ring).start()
        for i in range(nb):
            k = i % ring
            in_copy(i, k).wait()
            if i >= ring:
                out_copy(i - ring, k).wait()
            obuf[k] = xbuf[k] + bias
            out_copy(i, k).start()
            if i + ring < nb:
                in_copy(i + ring, k).start()
        for i in range(max(nb - ring, 0), nb):
            out_copy(i, i % ring).wait()

    return _body


@functools.partial(jax.jit, static_argnames=("bb", "ring"))
def _run(x, ts_i, ch_i, patch_pos_w, ch_pos_w, bb=16, ring=8):
    bs, max_c, max_n, emb = x.shape
    bias_fn = pl.kernel(
        _make_bias_body(max_c, max_n, emb),
        out_type=jax.ShapeDtypeStruct((max_c, max_n, emb), x.dtype),
        mesh=plsc.VectorSubcoreMesh(core_axis_name="c", subcore_axis_name="s"),
        scratch_types=[
            pltpu.VMEM((_L,), jnp.int32),
            pltpu.VMEM((2 * _L,), jnp.int32),
            pltpu.VMEM((max_n, emb), x.dtype),
            pltpu.VMEM((max_c, emb), x.dtype),
            pltpu.VMEM((max_n, emb), x.dtype),
        ],
    )
    bias = bias_fn(ts_i, ch_i, patch_pos_w, ch_pos_w)
    out = pl.pallas_call(
        _make_stream_body(bs, bb, ring),
        in_specs=[
            pl.BlockSpec(memory_space=pltpu.VMEM),
            pl.BlockSpec(memory_space=pl.ANY),
        ],
        out_specs=pl.BlockSpec(memory_space=pl.ANY),
        out_shape=jax.ShapeDtypeStruct((bs, max_c, max_n, emb), x.dtype),
        scratch_shapes=[
            pltpu.VMEM((ring, bb, max_c, max_n, emb), x.dtype),
            pltpu.VMEM((ring, bb, max_c, max_n, emb), x.dtype),
            pltpu.SemaphoreType.DMA((ring,)),
            pltpu.SemaphoreType.DMA((ring,)),
        ],
    )(bias, x)
    return out


def kernel(x, ts_token_mask, ch_mask, patch_pos_w, ch_pos_w):
    ts_i = jnp.pad(ts_token_mask.astype(jnp.int32),
                   ((0, 0), (0, _L - ts_token_mask.shape[1]))).reshape(_L)
    ch_i = jnp.pad(ch_mask.astype(jnp.int32),
                   ((0, 0), (0, 2 * _L - ch_mask.shape[1]))).reshape(2 * _L)
    return _run(x, ts_i, ch_i, patch_pos_w, ch_pos_w)
